# Initial kernel scaffold; baseline (speedup 1.0000x reference)
import jax
import jax.numpy as jnp
from jax.experimental import pallas as pl


def kernel(window, params):
    raise NotImplementedError("write your pallas kernel here")


# speedup vs baseline: 25.6623x; 25.6623x over previous
import functools
import math

import jax
import jax.numpy as jnp
from jax.experimental import pallas as pl

T = 2048
K = 16
QB = 256
WB = QB + 2 * K
NBLK = T // QB
HIDDEN = 128
NHEAD_T = 4
DH = HIDDEN // NHEAD_T
FF = 512
NEG = -1e30


def _leaky_relu(x):
    return jnp.where(x >= 0, x, 0.2 * x)


def _gat_body(xp_ref, w_ref, asrc_ref, adst_ref, b_ref, o_ref, *, heads, outd, act):
    i = pl.program_id(0)
    win = xp_ref[pl.ds(i * QB, WB), :]
    h_win = jnp.dot(win, w_ref[:, :], preferred_element_type=jnp.float32)
    iq = jax.lax.broadcasted_iota(jnp.int32, (QB, WB), 0)
    jk = jax.lax.broadcasted_iota(jnp.int32, (QB, WB), 1)
    g = i * QB + jk - K
    mask = (jk - iq >= 0) & (jk - iq <= 2 * K) & (g >= 0) & (g < T)
    outs = []
    for hd in range(heads):
        hh = h_win[:, hd * outd:(hd + 1) * outd]
        a_s = jnp.dot(hh, asrc_ref[hd, :][:, None],
                      preferred_element_type=jnp.float32)[:, 0]
        a_t = jnp.dot(hh, adst_ref[hd, :][:, None],
                      preferred_element_type=jnp.float32)[K:K + QB, 0]
        e = _leaky_relu(a_s[None, :] + a_t[:, None])
        e = jnp.where(mask, e, NEG)
        m = jnp.max(e, axis=1, keepdims=True)
        w = jnp.exp(e - m)
        w = jnp.where(mask, w, 0.0)
        z = jnp.sum(w, axis=1, keepdims=True)
        alpha = w / (z + 1e-16)
        outs.append(jnp.dot(alpha, hh, preferred_element_type=jnp.float32))
    out = jnp.concatenate(outs, axis=1) + b_ref[:]
    if act:
        out = jnp.maximum(out, 0.0)
    o_ref[:, :] = out


def _gat_layer(x, p, heads, outd, act):
    in_ch = x.shape[1]
    xp = jnp.pad(x, ((K, K), (0, 0)))
    body = functools.partial(_gat_body, heads=heads, outd=outd, act=act)
    return pl.pallas_call(
        body,
        grid=(NBLK,),
        in_specs=[
            pl.BlockSpec((T + 2 * K, in_ch), lambda i: (0, 0)),
            pl.BlockSpec((in_ch, heads * outd), lambda i: (0, 0)),
            pl.BlockSpec((heads, outd), lambda i: (0, 0)),
            pl.BlockSpec((heads, outd), lambda i: (0, 0)),
            pl.BlockSpec((heads * outd,), lambda i: (0,)),
        ],
        out_specs=pl.BlockSpec((QB, heads * outd), lambda i: (i, 0)),
        out_shape=jax.ShapeDtypeStruct((T, heads * outd), jnp.float32),
    )(xp, p["W"], p["att_src"], p["att_dst"], p["b"])


def _qkv_body(x_ref, w_ref, b_ref, o_ref):
    o_ref[:, :] = jnp.dot(x_ref[:, :], w_ref[:, :],
                          preferred_element_type=jnp.float32) + b_ref[:]


def _ln(x, w, b):
    mu = jnp.mean(x, axis=-1, keepdims=True)
    var = jnp.mean((x - mu) ** 2, axis=-1, keepdims=True)
    return (x - mu) * jax.lax.rsqrt(var + 1e-5) * w + b


def _tblock_body(x_ref, qkv_ref, wo_ref, bo_ref, ln1w_ref, ln1b_ref,
                 w1_ref, b1_ref, w2_ref, b2_ref, ln2w_ref, ln2b_ref, o_ref):
    i = pl.program_id(0)
    xb = x_ref[pl.ds(i * QB, QB), :]
    scale = 1.0 / math.sqrt(DH)
    outs = []
    for hd in range(NHEAD_T):
        qh = qkv_ref[pl.ds(i * QB, QB), hd * DH:(hd + 1) * DH]
        kh = qkv_ref[:, HIDDEN + hd * DH:HIDDEN + (hd + 1) * DH]
        vh = qkv_ref[:, 2 * HIDDEN + hd * DH:2 * HIDDEN + (hd + 1) * DH]
        s = jax.lax.dot_general(qh, kh, (((1,), (1,)), ((), ())),
                                preferred_element_type=jnp.float32) * scale
        m = jnp.max(s, axis=1, keepdims=True)
        w = jnp.exp(s - m)
        a = w / jnp.sum(w, axis=1, keepdims=True)
        outs.append(jnp.dot(a, vh, preferred_element_type=jnp.float32))
    o = jnp.concatenate(outs, axis=1)
    o = jnp.dot(o, wo_ref[:, :], preferred_element_type=jnp.float32) + bo_ref[:]
    x1 = _ln(xb + o, ln1w_ref[:], ln1b_ref[:])
    f = jnp.maximum(jnp.dot(x1, w1_ref[:, :],
                            preferred_element_type=jnp.float32) + b1_ref[:], 0.0)
    f = jnp.dot(f, w2_ref[:, :], preferred_element_type=jnp.float32) + b2_ref[:]
    o_ref[:, :] = _ln(x1 + f, ln2w_ref[:], ln2b_ref[:])


def _tlayer(x, p):
    qkv = pl.pallas_call(
        _qkv_body,
        out_shape=jax.ShapeDtypeStruct((T, 3 * HIDDEN), jnp.float32),
    )(x, p["Wqkv"].T, p["bqkv"])
    full = lambda shape: pl.BlockSpec(shape, lambda i: tuple(0 for _ in shape))
    return pl.pallas_call(
        _tblock_body,
        grid=(NBLK,),
        in_specs=[
            full((T, HIDDEN)), full((T, 3 * HIDDEN)),
            full((HIDDEN, HIDDEN)), full((HIDDEN,)),
            full((HIDDEN,)), full((HIDDEN,)),
            full((HIDDEN, FF)), full((FF,)),
            full((FF, HIDDEN)), full((HIDDEN,)),
            full((HIDDEN,)), full((HIDDEN,)),
        ],
        out_specs=pl.BlockSpec((QB, HIDDEN), lambda i: (i, 0)),
        out_shape=jax.ShapeDtypeStruct((T, HIDDEN), jnp.float32),
    )(x, qkv, p["Wo"].T, p["bo"], p["ln1_w"], p["ln1_b"],
      p["W1"].T, p["b1"], p["W2"].T, p["b2"], p["ln2_w"], p["ln2_b"])


def _cls_body(h_ref, w1_ref, b1_ref, w2_ref, b2_ref, o_ref):
    hc = jnp.mean(h_ref[:, :], axis=0, keepdims=True)
    h1 = jnp.maximum(jnp.dot(hc, w1_ref[:, :],
                             preferred_element_type=jnp.float32) + b1_ref[:], 0.0)
    o_ref[:, :] = jnp.dot(h1, w2_ref[:, :],
                          preferred_element_type=jnp.float32) + b2_ref[:]


def kernel(window, params):
    h = window
    for p in params["enc"]:
        h = _gat_layer(h, p, 4, 32, act=True)
    for p in params["trans"]:
        h = _tlayer(h, p)
    c = params["cls"]
    logits = pl.pallas_call(
        _cls_body,
        out_shape=jax.ShapeDtypeStruct((1, 2), jnp.float32),
    )(h, c["W1"].T, c["b1"], c["W2"].T, c["b2"])[0]
    x = h
    for p in params["dec"][:-1]:
        x = _gat_layer(x, p, 4, 32, act=True)
    out = _gat_layer(x, params["dec"][-1], 1, 64, act=False)
    return (out.T, logits)


# speedup vs baseline: 145.5891x; 5.6733x over previous
import functools
import math

import jax
import jax.numpy as jnp
from jax.experimental import pallas as pl
from jax.experimental.pallas import tpu as pltpu

T = 2048
K = 16
QB = 256
WB = QB + 2 * K
NBLK = T // QB
HIDDEN = 128
NHEAD_T = 4
DH = HIDDEN // NHEAD_T
FF = 512
NEG = -1e30


def _gat_body(xa_ref, xb_ref, xta_ref, xtb_ref, w_ref, wsrc_ref, wdst_ref,
              b_ref, o_ref, win_ref, wint_ref, *, heads, outd, act):
    i = pl.program_id(0)
    win_ref[:QB, :] = xa_ref[:, :]
    win_ref[QB:, :] = xb_ref[:2 * K, :]
    wint_ref[:, :QB] = xta_ref[:, :]
    wint_ref[:, QB:] = xtb_ref[:, :2 * K]
    win = win_ref[:, :]
    h_win = jnp.dot(win, w_ref[:, :], preferred_element_type=jnp.float32)
    a_src = jnp.dot(wsrc_ref[:, :], wint_ref[:, :],
                    preferred_element_type=jnp.float32)
    iq = jax.lax.broadcasted_iota(jnp.int32, (QB, WB), 0)
    jk = jax.lax.broadcasted_iota(jnp.int32, (QB, WB), 1)
    g = i * QB + jk - K
    band = (jk - iq >= 0) & (jk - iq <= 2 * K) & (g >= 0) & (g < T)
    madd = jnp.where(band, 0.0, NEG)
    outs = []
    for hd in range(heads):
        hh = h_win[:, hd * outd:(hd + 1) * outd]
        a_t = jnp.dot(hh[K:K + QB, :], wdst_ref[hd, :][:, None],
                      preferred_element_type=jnp.float32)
        s = a_src[hd:hd + 1, :] + a_t
        e = jnp.maximum(s, 0.2 * s) + madd
        m = jnp.max(e, axis=1, keepdims=True)
        w = jnp.exp(e - m)
        z = jnp.sum(w, axis=1, keepdims=True)
        o = jnp.dot(w, hh, preferred_element_type=jnp.float32)
        outs.append(o * (1.0 / (z + 1e-16)))
    out = jnp.concatenate(outs, axis=1) + b_ref[:]
    if act:
        out = jnp.maximum(out, 0.0)
    o_ref[:, :] = out


def _gat_layer(x, xt, p, heads, outd, act):
    in_ch = x.shape[1]
    pad_back = (NBLK + 1) * QB - T - K
    xp = jnp.pad(x, ((K, pad_back), (0, 0)))
    xtp = jnp.pad(xt, ((0, 0), (K, pad_back)))
    h_tot = heads * outd
    w3 = p["W"].reshape(in_ch, heads, outd)
    wsrc = jnp.einsum("iho,ho->hi", w3, p["att_src"])
    body = functools.partial(_gat_body, heads=heads, outd=outd, act=act)
    out = pl.pallas_call(
        body,
        grid=(NBLK,),
        in_specs=[
            pl.BlockSpec((QB, in_ch), lambda i: (i, 0)),
            pl.BlockSpec((QB, in_ch), lambda i: (i + 1, 0)),
            pl.BlockSpec((in_ch, QB), lambda i: (0, i)),
            pl.BlockSpec((in_ch, QB), lambda i: (0, i + 1)),
            pl.BlockSpec((in_ch, h_tot), lambda i: (0, 0)),
            pl.BlockSpec((heads, in_ch), lambda i: (0, 0)),
            pl.BlockSpec((heads, outd), lambda i: (0, 0)),
            pl.BlockSpec((h_tot,), lambda i: (0,)),
        ],
        out_specs=pl.BlockSpec((QB, h_tot), lambda i: (i, 0)),
        out_shape=jax.ShapeDtypeStruct((T, h_tot), jnp.float32),
        scratch_shapes=[pltpu.VMEM((WB, in_ch), jnp.float32),
                        pltpu.VMEM((in_ch, WB), jnp.float32)],
    )(xp, xp, xtp, xtp, p["W"], wsrc, p["att_dst"], p["b"])
    return out


def _qkv_body(x_ref, w_ref, b_ref, o_ref):
    o_ref[:, :] = jnp.dot(x_ref[:, :], w_ref[:, :],
                          preferred_element_type=jnp.float32) + b_ref[:]


def _ln(x, w, b):
    mu = jnp.mean(x, axis=-1, keepdims=True)
    var = jnp.mean((x - mu) ** 2, axis=-1, keepdims=True)
    return (x - mu) * jax.lax.rsqrt(var + 1e-5) * w + b


def _tblock_body(xb_ref, qkv_ref, qb_ref, wo_ref, bo_ref, ln1w_ref, ln1b_ref,
                 w1_ref, b1_ref, w2_ref, b2_ref, ln2w_ref, ln2b_ref, o_ref):
    xb = xb_ref[:, :]
    scale = 1.0 / math.sqrt(DH)
    outs = []
    for hd in range(NHEAD_T):
        qh = qb_ref[:, hd * DH:(hd + 1) * DH] * scale
        kh = qkv_ref[:, HIDDEN + hd * DH:HIDDEN + (hd + 1) * DH]
        vh = qkv_ref[:, 2 * HIDDEN + hd * DH:2 * HIDDEN + (hd + 1) * DH]
        s = jax.lax.dot_general(qh, kh, (((1,), (1,)), ((), ())),
                                preferred_element_type=jnp.float32)
        m = jnp.max(s, axis=1, keepdims=True)
        w = jnp.exp(s - m)
        z = jnp.sum(w, axis=1, keepdims=True)
        o = jnp.dot(w, vh, preferred_element_type=jnp.float32)
        outs.append(o * (1.0 / z))
    o = jnp.concatenate(outs, axis=1)
    o = jnp.dot(o, wo_ref[:, :], preferred_element_type=jnp.float32) + bo_ref[:]
    x1 = _ln(xb + o, ln1w_ref[:], ln1b_ref[:])
    f = jnp.maximum(jnp.dot(x1, w1_ref[:, :],
                            preferred_element_type=jnp.float32) + b1_ref[:], 0.0)
    f = jnp.dot(f, w2_ref[:, :], preferred_element_type=jnp.float32) + b2_ref[:]
    o_ref[:, :] = _ln(x1 + f, ln2w_ref[:], ln2b_ref[:])


def _tlayer(x, p):
    qkv = pl.pallas_call(
        _qkv_body,
        out_shape=jax.ShapeDtypeStruct((T, 3 * HIDDEN), jnp.float32),
    )(x, p["Wqkv"].T, p["bqkv"])
    full = lambda shape: pl.BlockSpec(shape, lambda i: tuple(0 for _ in shape))
    return pl.pallas_call(
        _tblock_body,
        grid=(NBLK,),
        in_specs=[
            pl.BlockSpec((QB, HIDDEN), lambda i: (i, 0)),
            full((T, 3 * HIDDEN)),
            pl.BlockSpec((QB, 3 * HIDDEN), lambda i: (i, 0)),
            full((HIDDEN, HIDDEN)), full((HIDDEN,)),
            full((HIDDEN,)), full((HIDDEN,)),
            full((HIDDEN, FF)), full((FF,)),
            full((FF, HIDDEN)), full((HIDDEN,)),
            full((HIDDEN,)), full((HIDDEN,)),
        ],
        out_specs=pl.BlockSpec((QB, HIDDEN), lambda i: (i, 0)),
        out_shape=jax.ShapeDtypeStruct((T, HIDDEN), jnp.float32),
    )(x, qkv, qkv, p["Wo"].T, p["bo"], p["ln1_w"], p["ln1_b"],
      p["W1"].T, p["b1"], p["W2"].T, p["b2"], p["ln2_w"], p["ln2_b"])


def _cls_body(h_ref, w1_ref, b1_ref, w2_ref, b2_ref, o_ref):
    hc = jnp.mean(h_ref[:, :], axis=0, keepdims=True)
    h1 = jnp.maximum(jnp.dot(hc, w1_ref[:, :],
                             preferred_element_type=jnp.float32) + b1_ref[:], 0.0)
    o_ref[:, :] = jnp.dot(h1, w2_ref[:, :],
                          preferred_element_type=jnp.float32) + b2_ref[:]


def kernel(window, params):
    h = window
    for p in params["enc"]:
        h = _gat_layer(h, h.T, p, 4, 32, act=True)
    for p in params["trans"]:
        h = _tlayer(h, p)
    c = params["cls"]
    logits = pl.pallas_call(
        _cls_body,
        out_shape=jax.ShapeDtypeStruct((1, 2), jnp.float32),
    )(h, c["W1"].T, c["b1"], c["W2"].T, c["b2"])[0]
    x = h
    for p in params["dec"][:-1]:
        x = _gat_layer(x, x.T, p, 4, 32, act=True)
    out = _gat_layer(x, x.T, params["dec"][-1], 1, 64, act=False)
    return (out.T, logits)


# speedup vs baseline: 171.4093x; 1.1773x over previous
import functools
import math

import jax
import jax.numpy as jnp
from jax.experimental import pallas as pl
from jax.experimental.pallas import tpu as pltpu

T = 2048
K = 16
QB = 256
WB = QB + 2 * K
NBLK = T // QB
HIDDEN = 128
NHEAD_T = 4
DH = HIDDEN // NHEAD_T
FF = 512
NEG = -1e30


def _gat_body(xm_ref, x0_ref, xp_ref, w_ref, wsrc_ref, wdst_ref,
              b_ref, o_ref, win_ref, *, heads, outd, act):
    i = pl.program_id(0)
    win_ref[:K, :] = xm_ref[QB - K:, :]
    win_ref[K:K + QB, :] = x0_ref[:, :]
    win_ref[K + QB:, :] = xp_ref[:K, :]
    win = win_ref[:, :]
    h_win = jnp.dot(win, w_ref[:, :], preferred_element_type=jnp.float32)
    a_src = jax.lax.dot_general(wsrc_ref[:, :], win, (((1,), (1,)), ((), ())),
                                preferred_element_type=jnp.float32)
    a_dst = jnp.dot(x0_ref[:, :], wdst_ref[:, :],
                    preferred_element_type=jnp.float32)
    iq = jax.lax.broadcasted_iota(jnp.int32, (QB, WB), 0)
    jk = jax.lax.broadcasted_iota(jnp.int32, (QB, WB), 1)
    g = i * QB + jk - K
    band = (jk - iq >= 0) & (jk - iq <= 2 * K) & (g >= 0) & (g < T)
    madd = jnp.where(band, 0.0, NEG)
    outs = []
    for hd in range(heads):
        hh = h_win[:, hd * outd:(hd + 1) * outd]
        s = a_src[hd:hd + 1, :] + a_dst[:, hd:hd + 1]
        e = jnp.maximum(s, 0.2 * s) + madd
        m = jnp.max(e, axis=1, keepdims=True)
        w = jnp.exp(e - m)
        z = jnp.sum(w, axis=1, keepdims=True)
        o = jnp.dot(w, hh, preferred_element_type=jnp.float32)
        outs.append(o * (1.0 / (z + 1e-16)))
    out = jnp.concatenate(outs, axis=1) + b_ref[:]
    if act:
        out = jnp.maximum(out, 0.0)
    o_ref[:, :] = out


def _gat_layer(x, p, heads, outd, act):
    in_ch = x.shape[1]
    h_tot = heads * outd
    w3 = p["W"].reshape(in_ch, heads, outd)
    wsrc = jnp.einsum("iho,ho->hi", w3, p["att_src"])
    wdst = jnp.einsum("iho,ho->ih", w3, p["att_dst"])
    body = functools.partial(_gat_body, heads=heads, outd=outd, act=act)
    out = pl.pallas_call(
        body,
        grid=(NBLK,),
        in_specs=[
            pl.BlockSpec((QB, in_ch), lambda i: (jnp.maximum(i - 1, 0), 0)),
            pl.BlockSpec((QB, in_ch), lambda i: (i, 0)),
            pl.BlockSpec((QB, in_ch),
                         lambda i: (jnp.minimum(i + 1, NBLK - 1), 0)),
            pl.BlockSpec((in_ch, h_tot), lambda i: (0, 0)),
            pl.BlockSpec((heads, in_ch), lambda i: (0, 0)),
            pl.BlockSpec((in_ch, heads), lambda i: (0, 0)),
            pl.BlockSpec((h_tot,), lambda i: (0,)),
        ],
        out_specs=pl.BlockSpec((QB, h_tot), lambda i: (i, 0)),
        out_shape=jax.ShapeDtypeStruct((T, h_tot), jnp.float32),
        scratch_shapes=[pltpu.VMEM((WB, in_ch), jnp.float32)],
    )(x, x, x, p["W"], wsrc, wdst, p["b"])
    return out


def _qkv_body(x_ref, w_ref, b_ref, o_ref):
    o_ref[:, :] = jnp.dot(x_ref[:, :], w_ref[:, :],
                          preferred_element_type=jnp.float32) + b_ref[:]


def _ln(x, w, b):
    mu = jnp.mean(x, axis=-1, keepdims=True)
    var = jnp.mean((x - mu) ** 2, axis=-1, keepdims=True)
    return (x - mu) * jax.lax.rsqrt(var + 1e-5) * w + b


def _tblock_body(xb_ref, qkv_ref, qb_ref, wo_ref, bo_ref, ln1w_ref, ln1b_ref,
                 w1_ref, b1_ref, w2_ref, b2_ref, ln2w_ref, ln2b_ref, o_ref):
    xb = xb_ref[:, :]
    scale = 1.0 / math.sqrt(DH)
    outs = []
    for hd in range(NHEAD_T):
        qh = qb_ref[:, hd * DH:(hd + 1) * DH] * scale
        kh = qkv_ref[:, HIDDEN + hd * DH:HIDDEN + (hd + 1) * DH]
        vh = qkv_ref[:, 2 * HIDDEN + hd * DH:2 * HIDDEN + (hd + 1) * DH]
        s = jax.lax.dot_general(qh, kh, (((1,), (1,)), ((), ())),
                                preferred_element_type=jnp.float32)
        m = jnp.max(s, axis=1, keepdims=True)
        w = jnp.exp(s - m)
        z = jnp.sum(w, axis=1, keepdims=True)
        o = jnp.dot(w, vh, preferred_element_type=jnp.float32)
        outs.append(o * (1.0 / z))
    o = jnp.concatenate(outs, axis=1)
    o = jnp.dot(o, wo_ref[:, :], preferred_element_type=jnp.float32) + bo_ref[:]
    x1 = _ln(xb + o, ln1w_ref[:], ln1b_ref[:])
    f = jnp.maximum(jnp.dot(x1, w1_ref[:, :],
                            preferred_element_type=jnp.float32) + b1_ref[:], 0.0)
    f = jnp.dot(f, w2_ref[:, :], preferred_element_type=jnp.float32) + b2_ref[:]
    o_ref[:, :] = _ln(x1 + f, ln2w_ref[:], ln2b_ref[:])


def _tlayer(x, p):
    qkv = pl.pallas_call(
        _qkv_body,
        out_shape=jax.ShapeDtypeStruct((T, 3 * HIDDEN), jnp.float32),
    )(x, p["Wqkv"].T, p["bqkv"])
    full = lambda shape: pl.BlockSpec(shape, lambda i: tuple(0 for _ in shape))
    return pl.pallas_call(
        _tblock_body,
        grid=(NBLK,),
        in_specs=[
            pl.BlockSpec((QB, HIDDEN), lambda i: (i, 0)),
            full((T, 3 * HIDDEN)),
            pl.BlockSpec((QB, 3 * HIDDEN), lambda i: (i, 0)),
            full((HIDDEN, HIDDEN)), full((HIDDEN,)),
            full((HIDDEN,)), full((HIDDEN,)),
            full((HIDDEN, FF)), full((FF,)),
            full((FF, HIDDEN)), full((HIDDEN,)),
            full((HIDDEN,)), full((HIDDEN,)),
        ],
        out_specs=pl.BlockSpec((QB, HIDDEN), lambda i: (i, 0)),
        out_shape=jax.ShapeDtypeStruct((T, HIDDEN), jnp.float32),
    )(x, qkv, qkv, p["Wo"].T, p["bo"], p["ln1_w"], p["ln1_b"],
      p["W1"].T, p["b1"], p["W2"].T, p["b2"], p["ln2_w"], p["ln2_b"])


def _cls_body(h_ref, w1_ref, b1_ref, w2_ref, b2_ref, o_ref):
    hc = jnp.mean(h_ref[:, :], axis=0, keepdims=True)
    h1 = jnp.maximum(jnp.dot(hc, w1_ref[:, :],
                             preferred_element_type=jnp.float32) + b1_ref[:], 0.0)
    o_ref[:, :] = jnp.dot(h1, w2_ref[:, :],
                          preferred_element_type=jnp.float32) + b2_ref[:]


def kernel(window, params):
    h = window
    for p in params["enc"]:
        h = _gat_layer(h, p, 4, 32, act=True)
    for p in params["trans"]:
        h = _tlayer(h, p)
    c = params["cls"]
    logits = pl.pallas_call(
        _cls_body,
        out_shape=jax.ShapeDtypeStruct((1, 2), jnp.float32),
    )(h, c["W1"].T, c["b1"], c["W2"].T, c["b2"])[0]
    x = h
    for p in params["dec"][:-1]:
        x = _gat_layer(x, p, 4, 32, act=True)
    out = _gat_layer(x, params["dec"][-1], 1, 64, act=False)
    return (out.T, logits)


# speedup vs baseline: 176.8568x; 1.0318x over previous
import jax
import jax.numpy as jnp
from jax.experimental import pallas as pl
from jax.experimental.pallas import tpu as pltpu

T = 2048
K = 16
QB = 256
WB = QB + 2 * K
NBLK = T // QB
HIDDEN = 128
NHEAD_T = 4
DH = HIDDEN // NHEAD_T
FF = 512
NEG = -1e30

ENC1 = 0
ENC2 = ENC1 + NBLK
ENC3 = ENC2 + NBLK
QKV1 = ENC3 + NBLK
ATT1 = QKV1 + 1
QKV2 = ATT1 + NBLK
ATT2 = QKV2 + 1
CLS = ATT2 + NBLK
DEC1 = CLS + 1
DEC2 = DEC1 + NBLK
DEC3 = DEC2 + NBLK
NSTEP = DEC3 + NBLK

SCALE = 1.0 / (DH ** 0.5)


def _gat_compute(b, win, w_ref, wsrc_ref, wdst_ref, b_ref, heads, outd, act):
    h_win = jnp.dot(win, w_ref[:, :], preferred_element_type=jnp.float32)
    a_src = jax.lax.dot_general(wsrc_ref[:, :], win, (((1,), (1,)), ((), ())),
                                preferred_element_type=jnp.float32)
    a_dst = jnp.dot(win[K:K + QB, :], wdst_ref[:, :],
                    preferred_element_type=jnp.float32)
    iq = jax.lax.broadcasted_iota(jnp.int32, (QB, WB), 0)
    jk = jax.lax.broadcasted_iota(jnp.int32, (QB, WB), 1)
    g = b * QB + jk - K
    band = (jk - iq >= 0) & (jk - iq <= 2 * K) & (g >= 0) & (g < T)
    madd = jnp.where(band, 0.0, NEG)
    outs = []
    for hd in range(heads):
        hh = h_win[:, hd * outd:(hd + 1) * outd]
        s = a_src[hd:hd + 1, :] + a_dst[:, hd:hd + 1]
        e = jnp.maximum(s, 0.2 * s) + madd
        m = jnp.max(e, axis=1, keepdims=True)
        w = jnp.exp(e - m)
        z = jnp.sum(w, axis=1, keepdims=True)
        o = jnp.dot(w, hh, preferred_element_type=jnp.float32)
        outs.append(o * (1.0 / (z + 1e-16)))
    out = jnp.concatenate(outs, axis=1) + b_ref[:]
    if act:
        out = jnp.maximum(out, 0.0)
    return out


def _ln(x, w, b):
    mu = jnp.mean(x, axis=-1, keepdims=True)
    var = jnp.mean((x - mu) ** 2, axis=-1, keepdims=True)
    return (x - mu) * jax.lax.rsqrt(var + 1e-5) * w + b


def _att_compute(b, src, qkv_ref, wo_ref, bo_ref, l1w, l1b, w1, b1, w2, b2,
                 l2w, l2b):
    xb = src[pl.ds(K + b * QB, QB), :]
    outs = []
    for hd in range(NHEAD_T):
        qh = qkv_ref[pl.ds(b * QB, QB), hd * DH:(hd + 1) * DH] * SCALE
        kh = qkv_ref[:, HIDDEN + hd * DH:HIDDEN + (hd + 1) * DH]
        vh = qkv_ref[:, 2 * HIDDEN + hd * DH:2 * HIDDEN + (hd + 1) * DH]
        s = jax.lax.dot_general(qh, kh, (((1,), (1,)), ((), ())),
                                preferred_element_type=jnp.float32)
        m = jnp.max(s, axis=1, keepdims=True)
        w = jnp.exp(s - m)
        z = jnp.sum(w, axis=1, keepdims=True)
        o = jnp.dot(w, vh, preferred_element_type=jnp.float32)
        outs.append(o * (1.0 / z))
    o = jnp.concatenate(outs, axis=1)
    o = jnp.dot(o, wo_ref[:, :], preferred_element_type=jnp.float32) + bo_ref[:]
    x1 = _ln(xb + o, l1w[:], l1b[:])
    f = jnp.maximum(jnp.dot(x1, w1[:, :],
                            preferred_element_type=jnp.float32) + b1[:], 0.0)
    f = jnp.dot(f, w2[:, :], preferred_element_type=jnp.float32) + b2[:]
    return _ln(x1 + f, l2w[:], l2b[:])


def _mega_body(xm_ref, x0_ref, xp_ref,
               w_e1, s_e1, d_e1, be1, w_e2, s_e2, d_e2, be2,
               w_e3, s_e3, d_e3, be3,
               qt1, bq1, wo1, bo1, aw1, ab1, f1a, f1ab, f1b, f1bb, cw1w, cb1w,
               qt2, bq2, wo2, bo2, aw2, ab2, f2a, f2ab, f2b, f2bb, cw2w, cb2w,
               clw1, clb1, clw2, clb2,
               w_d1, s_d1, d_d1, bd1, w_d2, s_d2, d_d2, bd2,
               w_d3, s_d3, d_d3, bd3,
               out_ref, logit_ref,
               bufA, bufB, qkvS, winS):
    i = pl.program_id(0)

    @pl.when(i == 0)
    def _():
        z = jnp.zeros((K, HIDDEN), jnp.float32)
        bufA[:K, :] = z
        bufA[K + T:, :] = z
        bufB[:K, :] = z
        bufB[K + T:, :] = z

    @pl.when(i < ENC2)
    def _():
        winS[:K, :] = xm_ref[QB - K:, :]
        winS[K:K + QB, :] = x0_ref[:, :]
        winS[K + QB:, :] = xp_ref[:K, :]
        out = _gat_compute(i, winS[:, :], w_e1, s_e1, d_e1, be1, 4, 32, True)
        bufA[pl.ds(K + i * QB, QB), :] = out

    @pl.when((i >= ENC2) & (i < ENC3))
    def _():
        b = i - ENC2
        win = bufA[pl.ds(b * QB, WB), :]
        bufB[pl.ds(K + b * QB, QB), :] = _gat_compute(
            b, win, w_e2, s_e2, d_e2, be2, 4, 32, True)

    @pl.when((i >= ENC3) & (i < QKV1))
    def _():
        b = i - ENC3
        win = bufB[pl.ds(b * QB, WB), :]
        bufA[pl.ds(K + b * QB, QB), :] = _gat_compute(
            b, win, w_e3, s_e3, d_e3, be3, 4, 32, True)

    @pl.when(i == QKV1)
    def _():
        qkvS[:, :] = jnp.dot(bufA[K:K + T, :], qt1[:, :],
                             preferred_element_type=jnp.float32) + bq1[:]

    @pl.when((i >= ATT1) & (i < QKV2))
    def _():
        b = i - ATT1
        bufB[pl.ds(K + b * QB, QB), :] = _att_compute(
            b, bufA, qkvS, wo1, bo1, aw1, ab1, f1a, f1ab, f1b, f1bb,
            cw1w, cb1w)

    @pl.when(i == QKV2)
    def _():
        qkvS[:, :] = jnp.dot(bufB[K:K + T, :], qt2[:, :],
                             preferred_element_type=jnp.float32) + bq2[:]

    @pl.when((i >= ATT2) & (i < CLS))
    def _():
        b = i - ATT2
        bufA[pl.ds(K + b * QB, QB), :] = _att_compute(
            b, bufB, qkvS, wo2, bo2, aw2, ab2, f2a, f2ab, f2b, f2bb,
            cw2w, cb2w)

    @pl.when(i == CLS)
    def _():
        hc = jnp.mean(bufA[K:K + T, :], axis=0, keepdims=True)
        h1 = jnp.maximum(
            jnp.dot(hc, clw1[:, :], preferred_element_type=jnp.float32)
            + clb1[:], 0.0)
        logit_ref[:, :] = jnp.dot(
            h1, clw2[:, :], preferred_element_type=jnp.float32) + clb2[:]

    @pl.when((i >= DEC1) & (i < DEC2))
    def _():
        b = i - DEC1
        win = bufA[pl.ds(b * QB, WB), :]
        bufB[pl.ds(K + b * QB, QB), :] = _gat_compute(
            b, win, w_d1, s_d1, d_d1, bd1, 4, 32, True)

    @pl.when((i >= DEC2) & (i < DEC3))
    def _():
        b = i - DEC2
        win = bufB[pl.ds(b * QB, WB), :]
        bufA[pl.ds(K + b * QB, QB), :] = _gat_compute(
            b, win, w_d2, s_d2, d_d2, bd2, 4, 32, True)

    @pl.when(i >= DEC3)
    def _():
        b = i - DEC3
        win = bufA[pl.ds(b * QB, WB), :]
        out_ref[:, :] = _gat_compute(b, win, w_d3, s_d3, d_d3, bd3, 1, 64,
                                     False)


def _fold_gat(p, heads, outd):
    in_ch = p["W"].shape[0]
    w3 = p["W"].reshape(in_ch, heads, outd)
    wsrc = jnp.einsum("iho,ho->hi", w3, p["att_src"])
    wdst = jnp.einsum("iho,ho->ih", w3, p["att_dst"])
    return [p["W"], wsrc, wdst, p["b"]]


def kernel(window, params):
    x = window
    in_ch = x.shape[1]

    gats = []
    for p in params["enc"]:
        gats += _fold_gat(p, 4, 32)
    t_ops = []
    for p in params["trans"]:
        t_ops += [p["Wqkv"].T, p["bqkv"], p["Wo"].T, p["bo"],
                  p["ln1_w"], p["ln1_b"], p["W1"].T, p["b1"],
                  p["W2"].T, p["b2"], p["ln2_w"], p["ln2_b"]]
    c = params["cls"]
    cls_ops = [c["W1"].T, c["b1"], c["W2"].T, c["b2"]]
    decs = []
    for p in params["dec"][:-1]:
        decs += _fold_gat(p, 4, 32)
    decs += _fold_gat(params["dec"][-1], 1, 64)

    operands = [x, x, x] + gats + t_ops + cls_ops + decs

    def _full(a):
        shape = a.shape
        nd = len(shape)
        return pl.BlockSpec(shape, lambda i, _nd=nd: (0,) * _nd)

    in_specs = [
        pl.BlockSpec((QB, in_ch),
                     lambda i: (jnp.maximum(jnp.clip(i, 0, NBLK - 1) - 1, 0),
                                0)),
        pl.BlockSpec((QB, in_ch), lambda i: (jnp.clip(i, 0, NBLK - 1), 0)),
        pl.BlockSpec((QB, in_ch),
                     lambda i: (jnp.minimum(jnp.clip(i, 0, NBLK - 1) + 1,
                                            NBLK - 1), 0)),
    ] + [_full(a) for a in operands[3:]]

    out, logits = pl.pallas_call(
        _mega_body,
        grid=(NSTEP,),
        in_specs=in_specs,
        out_specs=[
            pl.BlockSpec((QB, 64), lambda i: (jnp.clip(i - DEC3, 0,
                                                       NBLK - 1), 0)),
            pl.BlockSpec((1, 2), lambda i: (0, 0)),
        ],
        out_shape=[
            jax.ShapeDtypeStruct((T, 64), jnp.float32),
            jax.ShapeDtypeStruct((1, 2), jnp.float32),
        ],
        scratch_shapes=[
            pltpu.VMEM((T + 2 * K, HIDDEN), jnp.float32),
            pltpu.VMEM((T + 2 * K, HIDDEN), jnp.float32),
            pltpu.VMEM((T, 3 * HIDDEN), jnp.float32),
            pltpu.VMEM((WB, in_ch), jnp.float32),
        ],
    )(*operands)
    return (out.T, logits[0])
